# baseline (speedup 1.0000x reference)
import functools

import jax
import jax.numpy as jnp
from jax import lax
from jax.experimental import pallas as pl
from jax.experimental.pallas import tpu as pltpu
from jax.experimental.pallas import tpu_sc as plsc

TOPK = 64
L = 16


def _encode_body(x_ref, w_ref, benc_ref, bdec_ref, out_ref):
    x = x_ref[...] - bdec_ref[...][None, :]
    pre = jax.lax.dot_general(
        x, w_ref[...], (((1,), (1,)), ((), ())),
        preferred_element_type=jnp.float32)
    out_ref[...] = jnp.maximum(pre + benc_ref[...][None, :], 0.0)


def _decode_body(m_ref, w_ref, bdec_ref, out_ref):
    kk = pl.program_id(1)

    @pl.when(kk == 0)
    def _init():
        out_ref[...] = jnp.broadcast_to(bdec_ref[...][None, :], out_ref.shape)

    out_ref[...] += jax.lax.dot_general(
        m_ref[...], w_ref[...], (((1,), (1,)), ((), ())),
        preferred_element_type=jnp.float32)


def _pcnt(mask):
    return jnp.sum(mask.astype(jnp.int32))


def _sc_mask_body(rows_w, nvec, k, post_hbm, out_hbm, row_in, out_buf,
                  cand, gt_idx, gt_val, eq_idx, usel, hist, sem_in, sem_out):
    wid = lax.axis_index("s") * 2 + lax.axis_index("c")
    base = wid * rows_w
    lanes = lax.iota(jnp.int32, L)
    ones = jnp.ones((L,), jnp.int32)
    zf = jnp.zeros((L,), jnp.float32)
    zi = jnp.zeros((L,), jnp.int32)

    def _zero(i, _):
        out_buf[0, pl.ds(i * L, L)] = zf
        out_buf[1, pl.ds(i * L, L)] = zf
        return 0
    lax.fori_loop(0, nvec, _zero, 0)

    def _zeroh(i, _):
        hist[pl.ds(i * L, L)] = zi
        return 0
    lax.fori_loop(0, 256, _zeroh, 0)

    pltpu.make_async_copy(post_hbm.at[base], row_in.at[0], sem_in.at[0]).start()
    pltpu.make_async_copy(post_hbm.at[base + 1], row_in.at[1],
                          sem_in.at[1]).start()

    def _process(r, ib):
        row = base + r
        pltpu.make_async_copy(post_hbm.at[row], row_in.at[ib],
                              sem_in.at[ib]).wait()
        ibv = jnp.full((L,), ib, jnp.int32)

        def _hist(i, _):
            bits = lax.bitcast_convert_type(row_in[ib, pl.ds(i * L, L)], jnp.int32)
            plsc.addupdate_scatter(hist, [lanes * 256 + (bits >> 23)], ones)
            return 0
        lax.fori_loop(0, nvec, _hist, 0)

        def _scan(cc, carry):
            tot_above, b = carry
            c = 15 - cc
            tot = jnp.zeros((L,), jnp.int32)
            for lane in range(L):
                off = lane * 256 + c * L
                tot = tot + hist[pl.ds(off, L)]
                hist[pl.ds(off, L)] = zi
            suf = lax.rev(plsc.cumsum(lax.rev(tot, (0,))), (0,)) + tot_above
            p = _pcnt(suf >= k) - 1
            b = jnp.where((b < 0) & (p >= 0), c * L + p, b)
            return tot_above + jnp.sum(tot), b
        _, b = lax.fori_loop(0, 16, _scan, (jnp.int32(0), jnp.int32(-1)))

        t0 = b << 23
        def _extract(i, cnt):
            bits = lax.bitcast_convert_type(row_in[ib, pl.ds(i * L, L)], jnp.int32)
            m = bits >= t0
            plsc.store_compressed(cand.at[pl.ds(cnt, L)], i * L + lanes, mask=m)
            return cnt + _pcnt(m)
        c_n = lax.fori_loop(0, nvec, _extract, jnp.int32(0))
        nv = (c_n + L - 1) // L

        def _count_ge(t):
            def _cnt(j, acc):
                valid = j * L + lanes < c_n
                idxv = jnp.where(valid, cand[pl.ds(j * L, L)], 0)
                vals = plsc.load_gather(row_in, [ibv, idxv], mask=valid)
                bits = lax.bitcast_convert_type(vals, jnp.int32)
                return acc + _pcnt((bits >= t) & valid)
            return lax.fori_loop(0, nv, _cnt, jnp.int32(0))

        def _bs(j, tlo):
            t = tlo | (jnp.int32(1) << (22 - j))
            return jnp.where(_count_ge(t) >= k, t, tlo)
        t_k = lax.fori_loop(0, 23, _bs, t0)

        def _sel(j, carry):
            g, e = carry
            valid = j * L + lanes < c_n
            idxv = jnp.where(valid, cand[pl.ds(j * L, L)], 0)
            vals = plsc.load_gather(row_in, [ibv, idxv], mask=valid)
            bits = lax.bitcast_convert_type(vals, jnp.int32)
            mgt = (bits > t_k) & valid
            meq = (bits == t_k) & valid
            plsc.store_compressed(gt_idx.at[pl.ds(g, L)], idxv, mask=mgt)
            plsc.store_compressed(gt_val.at[pl.ds(g, L)], vals, mask=mgt)
            plsc.store_compressed(eq_idx.at[pl.ds(jnp.minimum(e, 64), L)],
                                  idxv, mask=meq)
            return g + _pcnt(mgt), e + _pcnt(meq)
        g_n, _ = lax.fori_loop(0, nv, _sel, (jnp.int32(0), jnp.int32(0)))

        @pl.when(r >= 2)
        def _reclaim():
            pltpu.make_async_copy(out_buf.at[ib], out_hbm.at[row - 2],
                                  sem_out.at[ib]).wait()
            for j in range(TOPK // L):
                m = j * L + lanes < TOPK
                uv = jnp.where(m, usel[ib, pl.ds(j * L, L)], 0)
                plsc.store_scatter(out_buf, [ibv, uv], zf, mask=m)

        tf = lax.bitcast_convert_type(jnp.full((L,), t_k, jnp.int32), jnp.float32)
        for j in range(TOPK // L):
            pos = j * L + lanes
            mg = pos < g_n
            gi = jnp.where(mg, gt_idx[pl.ds(j * L, L)], 0)
            gv = gt_val[pl.ds(j * L, L)]
            plsc.store_scatter(out_buf, [ibv, gi], gv, mask=mg)
            me = pos < (k - g_n)
            ei = jnp.where(me, eq_idx[pl.ds(j * L, L)], 0)
            plsc.store_scatter(out_buf, [ibv, ei], tf, mask=me)
            usel[ib, pl.ds(j * L, L)] = gt_idx[pl.ds(j * L, L)]
        for j in range(TOPK // L):
            usel[ib, pl.ds(g_n + j * L, L)] = eq_idx[pl.ds(j * L, L)]

        pltpu.make_async_copy(out_buf.at[ib], out_hbm.at[row],
                              sem_out.at[ib]).start()

        @pl.when(r + 2 < rows_w)
        def _refill():
            pltpu.make_async_copy(post_hbm.at[row + 2], row_in.at[ib],
                                  sem_in.at[ib]).start()

    def _pair(q, _):
        _process(2 * q, 0)
        _process(2 * q + 1, 1)
        return 0
    lax.fori_loop(0, rows_w // 2, _pair, 0)

    pltpu.make_async_copy(out_buf.at[0], out_hbm.at[base + rows_w - 2],
                          sem_out.at[0]).wait()
    pltpu.make_async_copy(out_buf.at[1], out_hbm.at[base + rows_w - 1],
                          sem_out.at[1]).wait()


def _sc_topk_mask(post, k):
    B, F = post.shape
    info = plsc.get_sparse_core_info()
    nw = info.num_cores * info.num_subcores
    rows_w = B // nw
    nvec = F // L
    mesh = plsc.VectorSubcoreMesh(core_axis_name="c", subcore_axis_name="s")
    fn = pl.kernel(
        functools.partial(_sc_mask_body, rows_w, nvec, k),
        out_type=jax.ShapeDtypeStruct((B, F), jnp.float32),
        mesh=mesh,
        compiler_params=pltpu.CompilerParams(needs_layout_passes=False),
        scratch_types=[
            pltpu.VMEM((2, F), jnp.float32),
            pltpu.VMEM((2, F), jnp.float32),
            pltpu.VMEM((F + 2 * L,), jnp.int32),
            pltpu.VMEM((96,), jnp.int32),
            pltpu.VMEM((96,), jnp.float32),
            pltpu.VMEM((96,), jnp.int32),
            pltpu.VMEM((2, 160), jnp.int32),
            pltpu.VMEM((16 * 256,), jnp.int32),
            pltpu.SemaphoreType.DMA((2,)),
            pltpu.SemaphoreType.DMA((2,)),
        ],
    )
    return fn(post)


def _run(x, W_enc, b_enc, W_dec, b_dec, *, k, tb, tf, tb3, tfk,
         interpret=False):
    B, D = x.shape
    F = W_enc.shape[0]

    post = pl.pallas_call(
        _encode_body,
        grid=(F // tf, B // tb),
        in_specs=[
            pl.BlockSpec((tb, D), lambda f, b: (b, 0)),
            pl.BlockSpec((tf, D), lambda f, b: (f, 0)),
            pl.BlockSpec((tf,), lambda f, b: (f,)),
            pl.BlockSpec((D,), lambda f, b: (0,)),
        ],
        out_specs=pl.BlockSpec((tb, tf), lambda f, b: (b, f)),
        out_shape=jax.ShapeDtypeStruct((B, F), jnp.float32),
        interpret=interpret,
    )(x, W_enc, b_enc, b_dec)

    masked = _sc_topk_mask(post, k)

    x_hat = pl.pallas_call(
        _decode_body,
        grid=(B // tb3, F // tfk),
        in_specs=[
            pl.BlockSpec((tb3, tfk), lambda i, kk: (i, kk)),
            pl.BlockSpec((D, tfk), lambda i, kk: (0, kk)),
            pl.BlockSpec((D,), lambda i, kk: (0,)),
        ],
        out_specs=pl.BlockSpec((tb3, D), lambda i, kk: (i, 0)),
        out_shape=jax.ShapeDtypeStruct((B, D), jnp.float32),
        compiler_params=pltpu.CompilerParams(
            dimension_semantics=("arbitrary", "arbitrary")),
        interpret=interpret,
    )(masked, W_dec, b_dec)
    return x_hat


def kernel(x, W_enc, b_enc, W_dec, b_dec):
    return _run(x, W_enc, b_enc, W_dec, b_dec,
                k=TOPK, tb=512, tf=2048, tb3=512, tfk=2048)


# speedup vs baseline: 1.4113x; 1.4113x over previous
import functools

import jax
import jax.numpy as jnp
from jax import lax
from jax.experimental import pallas as pl
from jax.experimental.pallas import tpu as pltpu
from jax.experimental.pallas import tpu_sc as plsc

TOPK = 64
L = 16


def _encode_body(x_ref, w_ref, benc_ref, bdec_ref, out_ref):
    x = x_ref[...] - bdec_ref[...][None, :]
    pre = jax.lax.dot_general(
        x, w_ref[...], (((1,), (1,)), ((), ())),
        preferred_element_type=jnp.float32)
    out_ref[...] = jnp.maximum(pre + benc_ref[...][None, :], 0.0)


def _decode_body(m_ref, w_ref, bdec_ref, out_ref):
    kk = pl.program_id(1)

    @pl.when(kk == 0)
    def _init():
        out_ref[...] = jnp.broadcast_to(bdec_ref[...][None, :], out_ref.shape)

    out_ref[...] += jax.lax.dot_general(
        m_ref[...], w_ref[...], (((1,), (1,)), ((), ())),
        preferred_element_type=jnp.float32)


def _pcnt(mask):
    return jnp.sum(mask.astype(jnp.int32))


def _sc_mask_body(rows_w, nvec, k, post_hbm, out_hbm, row_in, out_buf,
                  cand, gt_idx, gt_val, eq_idx, usel, hist, tmp,
                  sem_in, sem_out):
    wid = lax.axis_index("s") * 2 + lax.axis_index("c")
    base = wid * rows_w
    lanes = lax.iota(jnp.int32, L)
    ones = jnp.ones((L,), jnp.int32)
    zf = jnp.zeros((L,), jnp.float32)
    zi = jnp.zeros((L,), jnp.int32)
    UN = 8

    def _zero(i, _):
        out_buf[0, pl.ds(i * L, L)] = zf
        out_buf[1, pl.ds(i * L, L)] = zf
        return 0
    lax.fori_loop(0, nvec, _zero, 0)

    def _zeroh(i, _):
        hist[pl.ds(i * L, L)] = zi
        return 0
    lax.fori_loop(0, 256, _zeroh, 0)

    pltpu.make_async_copy(post_hbm.at[base], row_in.at[0], sem_in.at[0]).start()
    pltpu.make_async_copy(post_hbm.at[base + 1], row_in.at[1],
                          sem_in.at[1]).start()

    def _scalar(v):
        return v[0]

    def _process(r, ib, tg):
        row = base + r
        pltpu.make_async_copy(post_hbm.at[row], row_in.at[ib],
                              sem_in.at[ib]).wait()
        ibv = jnp.full((L,), ib, jnp.int32)

        def _extract(t0s):
            def _ex(i, cnt):
                for u in range(UN):
                    ii = i * UN + u
                    bits = lax.bitcast_convert_type(
                        row_in[ib, pl.ds(ii * L, L)], jnp.int32)
                    m = bits >= t0s
                    dest = cnt + plsc.cumsum(m.astype(jnp.int32)) - 1
                    plsc.store_scatter(cand, [dest], ii * L + lanes, mask=m)
                    cnt = cnt + plsc.all_reduce_population_count(m)
                return cnt
            return lax.fori_loop(0, nvec // UN, _ex, zi)

        c_n_v = _extract(tg)
        c_n0 = _scalar(c_n_v)

        def _fallback(_):
            def _hist(i, _2):
                for u in range(UN):
                    ii = i * UN + u
                    bits = lax.bitcast_convert_type(
                        row_in[ib, pl.ds(ii * L, L)], jnp.int32)
                    plsc.addupdate_scatter(
                        hist, [lanes * 256 + (bits >> 23)], ones)
                return 0
            lax.fori_loop(0, nvec // UN, _hist, 0)

            def _scan(cc, carry):
                tot_above, b = carry
                c = 15 - cc
                tot = zi
                for lane in range(L):
                    off = lane * 256 + c * L
                    tot = tot + hist[pl.ds(off, L)]
                    hist[pl.ds(off, L)] = zi
                suf = lax.rev(plsc.cumsum(lax.rev(tot, (0,))), (0,)) + tot_above
                p = jnp.sum((suf >= k).astype(jnp.int32)) - 1
                b = jnp.where((b < 0) & (p >= 0), c * L + p, b)
                return tot_above + jnp.sum(tot), b
            _, b = lax.fori_loop(0, 16, _scan, (jnp.int32(0), jnp.int32(-1)))
            return _extract(jnp.full((L,), b << 23, jnp.int32))

        c_n_v = lax.cond((c_n0 < k) | (c_n0 > 1024), _fallback,
                         lambda _: c_n_v, 0)
        c_n = _scalar(c_n_v)
        nv = (c_n + L - 1) // L

        def _count_ge(t):
            def _cnt(j, acc):
                valid = j * L + lanes < c_n_v
                idxv = jnp.where(valid, cand[pl.ds(j * L, L)], 0)
                vals = plsc.load_gather(row_in, [ibv, idxv], mask=valid)
                bits = lax.bitcast_convert_type(vals, jnp.int32)
                return acc + plsc.all_reduce_population_count(
                    (bits >= t) & valid)
            return lax.fori_loop(0, nv, _cnt, zi)

        tlo = zi
        for bit in range(30, -1, -1):
            t = tlo | (1 << bit)
            cnt = _count_ge(t)
            tlo = jnp.where(cnt >= k, t, tlo)
        t_k = tlo

        def _sel(j, carry):
            g, e = carry
            valid = j * L + lanes < c_n_v
            idxv = jnp.where(valid, cand[pl.ds(j * L, L)], 0)
            vals = plsc.load_gather(row_in, [ibv, idxv], mask=valid)
            bits = lax.bitcast_convert_type(vals, jnp.int32)
            mgt = (bits > t_k) & valid
            meq = (bits == t_k) & valid
            destg = g + plsc.cumsum(mgt.astype(jnp.int32)) - 1
            plsc.store_scatter(gt_idx, [destg], idxv, mask=mgt)
            plsc.store_scatter(gt_val, [destg], vals, mask=mgt)
            deste = jnp.minimum(
                e + plsc.cumsum(meq.astype(jnp.int32)) - 1, 72 + lanes)
            plsc.store_scatter(eq_idx, [deste], idxv, mask=meq)
            return (g + plsc.all_reduce_population_count(mgt),
                    e + plsc.all_reduce_population_count(meq))
        g_v, _ = lax.fori_loop(0, nv, _sel, (zi, zi))
        g_n = _scalar(g_v)

        @pl.when(r >= 2)
        def _reclaim():
            pltpu.make_async_copy(out_buf.at[ib], out_hbm.at[row - 2],
                                  sem_out.at[ib]).wait()
            for j in range(TOPK // L):
                m = j * L + lanes < TOPK
                uv = jnp.where(m, usel[ib, pl.ds(j * L, L)], 0)
                plsc.store_scatter(out_buf, [ibv, uv], zf, mask=m)

        tf = lax.bitcast_convert_type(t_k, jnp.float32)
        for j in range(TOPK // L):
            pos = j * L + lanes
            mg = pos < g_v
            gi = jnp.where(mg, gt_idx[pl.ds(j * L, L)], 0)
            gv = gt_val[pl.ds(j * L, L)]
            plsc.store_scatter(out_buf, [ibv, gi], gv, mask=mg)
            me = pos < (k - g_v)
            ei = jnp.where(me, eq_idx[pl.ds(j * L, L)], 0)
            plsc.store_scatter(out_buf, [ibv, ei], tf, mask=me)
            usel[ib, pl.ds(j * L, L)] = gt_idx[pl.ds(j * L, L)]
        for j in range(TOPK // L):
            usel[ib, pl.ds(g_n + j * L, L)] = eq_idx[pl.ds(j * L, L)]

        pltpu.make_async_copy(out_buf.at[ib], out_hbm.at[row],
                              sem_out.at[ib]).start()

        @pl.when(r + 2 < rows_w)
        def _refill():
            pltpu.make_async_copy(post_hbm.at[row + 2], row_in.at[ib],
                                  sem_in.at[ib]).start()

        return jnp.maximum(t_k - (1 << 21), 0)

    def _pair(q, tg):
        tg = _process(2 * q, 0, tg)
        tg = _process(2 * q + 1, 1, tg)
        return tg
    lax.fori_loop(0, rows_w // 2, _pair, zi)

    pltpu.make_async_copy(out_buf.at[0], out_hbm.at[base + rows_w - 2],
                          sem_out.at[0]).wait()
    pltpu.make_async_copy(out_buf.at[1], out_hbm.at[base + rows_w - 1],
                          sem_out.at[1]).wait()


def _sc_topk_mask(post, k):
    B, F = post.shape
    info = plsc.get_sparse_core_info()
    nw = info.num_cores * info.num_subcores
    rows_w = B // nw
    nvec = F // L
    mesh = plsc.VectorSubcoreMesh(core_axis_name="c", subcore_axis_name="s")
    fn = pl.kernel(
        functools.partial(_sc_mask_body, rows_w, nvec, k),
        out_type=jax.ShapeDtypeStruct((B, F), jnp.float32),
        mesh=mesh,
        compiler_params=pltpu.CompilerParams(needs_layout_passes=False),
        scratch_types=[
            pltpu.VMEM((2, F), jnp.float32),
            pltpu.VMEM((2, F), jnp.float32),
            pltpu.VMEM((F + 2 * L,), jnp.int32),
            pltpu.VMEM((96,), jnp.int32),
            pltpu.VMEM((96,), jnp.float32),
            pltpu.VMEM((96,), jnp.int32),
            pltpu.VMEM((2, 160), jnp.int32),
            pltpu.VMEM((16 * 256,), jnp.int32),
            pltpu.VMEM((L,), jnp.int32),
            pltpu.SemaphoreType.DMA((2,)),
            pltpu.SemaphoreType.DMA((2,)),
        ],
    )
    return fn(post)


def _run(x, W_enc, b_enc, W_dec, b_dec, *, k, tb, tf, tb3, tfk,
         interpret=False):
    B, D = x.shape
    F = W_enc.shape[0]

    post = pl.pallas_call(
        _encode_body,
        grid=(F // tf, B // tb),
        in_specs=[
            pl.BlockSpec((tb, D), lambda f, b: (b, 0)),
            pl.BlockSpec((tf, D), lambda f, b: (f, 0)),
            pl.BlockSpec((tf,), lambda f, b: (f,)),
            pl.BlockSpec((D,), lambda f, b: (0,)),
        ],
        out_specs=pl.BlockSpec((tb, tf), lambda f, b: (b, f)),
        out_shape=jax.ShapeDtypeStruct((B, F), jnp.float32),
        interpret=interpret,
    )(x, W_enc, b_enc, b_dec)

    masked = _sc_topk_mask(post, k)

    x_hat = pl.pallas_call(
        _decode_body,
        grid=(B // tb3, F // tfk),
        in_specs=[
            pl.BlockSpec((tb3, tfk), lambda i, kk: (i, kk)),
            pl.BlockSpec((D, tfk), lambda i, kk: (0, kk)),
            pl.BlockSpec((D,), lambda i, kk: (0,)),
        ],
        out_specs=pl.BlockSpec((tb3, D), lambda i, kk: (i, 0)),
        out_shape=jax.ShapeDtypeStruct((B, D), jnp.float32),
        compiler_params=pltpu.CompilerParams(
            dimension_semantics=("arbitrary", "arbitrary")),
        interpret=interpret,
    )(masked, W_dec, b_dec)
    return x_hat


def kernel(x, W_enc, b_enc, W_dec, b_dec):
    return _run(x, W_enc, b_enc, W_dec, b_dec,
                k=TOPK, tb=512, tf=2048, tb3=512, tfk=2048)


# speedup vs baseline: 2.7218x; 1.9285x over previous
import functools

import jax
import jax.numpy as jnp
from jax import lax
from jax.experimental import pallas as pl
from jax.experimental.pallas import tpu as pltpu
from jax.experimental.pallas import tpu_sc as plsc

TOPK = 64
L = 16


def _encode_body(x_ref, w_ref, benc_ref, bdec_ref, out_ref):
    x = x_ref[...] - bdec_ref[...][None, :]
    pre = jax.lax.dot_general(
        x, w_ref[...], (((1,), (1,)), ((), ())),
        preferred_element_type=jnp.float32)
    out_ref[...] = jnp.maximum(pre + benc_ref[...][None, :], 0.0)


def _decode_body(m_ref, w_ref, bdec_ref, out_ref):
    kk = pl.program_id(1)

    @pl.when(kk == 0)
    def _init():
        out_ref[...] = jnp.broadcast_to(bdec_ref[...][None, :], out_ref.shape)

    out_ref[...] += jax.lax.dot_general(
        m_ref[...], w_ref[...], (((1,), (1,)), ((), ())),
        preferred_element_type=jnp.float32)


def _pcnt(mask):
    return jnp.sum(mask.astype(jnp.int32))


def _sc_mask_body(rows_w, nvec, k, post_hbm, out_hbm, row_in, out_buf,
                  cand, cand_val, gt_idx, gt_val, eq_idx, usel, hist, tmp,
                  sem_in, sem_out):
    wid = lax.axis_index("s") * 2 + lax.axis_index("c")
    base = wid * rows_w
    lanes = lax.iota(jnp.int32, L)
    ones = jnp.ones((L,), jnp.int32)
    zf = jnp.zeros((L,), jnp.float32)
    zi = jnp.zeros((L,), jnp.int32)
    UN = 8

    def _zero(i, _):
        out_buf[0, pl.ds(i * L, L)] = zf
        out_buf[1, pl.ds(i * L, L)] = zf
        return 0
    lax.fori_loop(0, nvec, _zero, 0)

    def _zeroh(i, _):
        hist[pl.ds(i * L, L)] = zi
        return 0
    lax.fori_loop(0, 256, _zeroh, 0)

    pltpu.make_async_copy(post_hbm.at[base], row_in.at[0], sem_in.at[0]).start()
    pltpu.make_async_copy(post_hbm.at[base + 1], row_in.at[1],
                          sem_in.at[1]).start()

    def _scalar(v):
        return v[0]

    def _process(r, ib, tg):
        row = base + r
        pltpu.make_async_copy(post_hbm.at[row], row_in.at[ib],
                              sem_in.at[ib]).wait()
        ibv = jnp.full((L,), ib, jnp.int32)

        def _extract(t0s):
            @plsc.parallel_loop(0, nvec, unroll=UN, carry=zi)
            def _ex(i, cnt):
                vals = row_in[ib, pl.ds(i * L, L)]
                bits = lax.bitcast_convert_type(vals, jnp.int32)
                m = bits >= t0s
                dest = cnt + plsc.cumsum(m.astype(jnp.int32)) - 1
                plsc.store_scatter(cand, [dest], i * L + lanes, mask=m)
                plsc.store_scatter(cand_val, [dest], vals, mask=m)
                return cnt + plsc.all_reduce_population_count(m)
            return _ex

        c_n_v = _extract(tg)
        c_n0 = _scalar(c_n_v)

        def _fallback(_):
            @plsc.parallel_loop(0, nvec, unroll=UN)
            def _hist(i):
                bits = lax.bitcast_convert_type(
                    row_in[ib, pl.ds(i * L, L)], jnp.int32)
                plsc.addupdate_scatter(
                    hist, [lanes * 256 + (bits >> 23)], ones)

            def _scan(cc, carry):
                tot_above, b = carry
                c = 15 - cc
                tot = zi
                for lane in range(L):
                    off = lane * 256 + c * L
                    tot = tot + hist[pl.ds(off, L)]
                    hist[pl.ds(off, L)] = zi
                suf = lax.rev(plsc.cumsum(lax.rev(tot, (0,))), (0,)) + tot_above
                p = jnp.sum((suf >= k).astype(jnp.int32)) - 1
                b = jnp.where((b < 0) & (p >= 0), c * L + p, b)
                return tot_above + jnp.sum(tot), b
            _, b = lax.fori_loop(0, 16, _scan, (jnp.int32(0), jnp.int32(-1)))
            return _extract(jnp.full((L,), b << 23, jnp.int32))

        c_n_v = lax.cond((c_n0 < k) | (c_n0 > 1024), _fallback,
                         lambda _: c_n_v, 0)
        c_n = _scalar(c_n_v)
        nv = (c_n + L - 1) // L
        cand[pl.ds(c_n, L)] = zi
        cand_val[pl.ds(c_n, L)] = zf

        def _count_ge(t):
            @plsc.parallel_loop(0, nv, unroll=4, carry=zi)
            def _cnt(j, acc):
                bits = lax.bitcast_convert_type(
                    cand_val[pl.ds(j * L, L)], jnp.int32)
                return acc + plsc.all_reduce_population_count(bits >= t)
            return _cnt

        tlo = zi
        for bit in range(30, -1, -1):
            t = tlo | (1 << bit)
            cnt = _count_ge(t)
            tlo = jnp.where(cnt >= k, t, tlo)
        t_k = tlo

        def _sel(j, carry):
            g, e = carry
            idxv = cand[pl.ds(j * L, L)]
            vals = cand_val[pl.ds(j * L, L)]
            bits = lax.bitcast_convert_type(vals, jnp.int32)
            mgt = bits > t_k
            meq = bits == t_k
            destg = g + plsc.cumsum(mgt.astype(jnp.int32)) - 1
            plsc.store_scatter(gt_idx, [destg], idxv, mask=mgt)
            plsc.store_scatter(gt_val, [destg], vals, mask=mgt)
            deste = jnp.minimum(
                e + plsc.cumsum(meq.astype(jnp.int32)) - 1, 72 + lanes)
            plsc.store_scatter(eq_idx, [deste], idxv, mask=meq)
            return (g + plsc.all_reduce_population_count(mgt),
                    e + plsc.all_reduce_population_count(meq))
        g_v, _ = lax.fori_loop(0, nv, _sel, (zi, zi))
        g_n = _scalar(g_v)

        @pl.when(r >= 2)
        def _reclaim():
            pltpu.make_async_copy(out_buf.at[ib], out_hbm.at[row - 2],
                                  sem_out.at[ib]).wait()
            for j in range(TOPK // L):
                m = j * L + lanes < TOPK
                uv = jnp.where(m, usel[ib, pl.ds(j * L, L)], 0)
                plsc.store_scatter(out_buf, [ibv, uv], zf, mask=m)

        tf = lax.bitcast_convert_type(t_k, jnp.float32)
        for j in range(TOPK // L):
            pos = j * L + lanes
            mg = pos < g_v
            gi = jnp.where(mg, gt_idx[pl.ds(j * L, L)], 0)
            gv = gt_val[pl.ds(j * L, L)]
            plsc.store_scatter(out_buf, [ibv, gi], gv, mask=mg)
            me = pos < (k - g_v)
            ei = jnp.where(me, eq_idx[pl.ds(j * L, L)], 0)
            plsc.store_scatter(out_buf, [ibv, ei], tf, mask=me)
            usel[ib, pl.ds(j * L, L)] = gt_idx[pl.ds(j * L, L)]
        for j in range(TOPK // L):
            usel[ib, pl.ds(g_n + j * L, L)] = eq_idx[pl.ds(j * L, L)]

        pltpu.make_async_copy(out_buf.at[ib], out_hbm.at[row],
                              sem_out.at[ib]).start()

        @pl.when(r + 2 < rows_w)
        def _refill():
            pltpu.make_async_copy(post_hbm.at[row + 2], row_in.at[ib],
                                  sem_in.at[ib]).start()

        return jnp.maximum(t_k - (1 << 21), 0)

    def _pair(q, tg):
        tg = _process(2 * q, 0, tg)
        tg = _process(2 * q + 1, 1, tg)
        return tg
    lax.fori_loop(0, rows_w // 2, _pair, zi)

    pltpu.make_async_copy(out_buf.at[0], out_hbm.at[base + rows_w - 2],
                          sem_out.at[0]).wait()
    pltpu.make_async_copy(out_buf.at[1], out_hbm.at[base + rows_w - 1],
                          sem_out.at[1]).wait()


def _sc_topk_mask(post, k):
    B, F = post.shape
    info = plsc.get_sparse_core_info()
    nw = info.num_cores * info.num_subcores
    rows_w = B // nw
    nvec = F // L
    mesh = plsc.VectorSubcoreMesh(core_axis_name="c", subcore_axis_name="s")
    fn = pl.kernel(
        functools.partial(_sc_mask_body, rows_w, nvec, k),
        out_type=jax.ShapeDtypeStruct((B, F), jnp.float32),
        mesh=mesh,
        compiler_params=pltpu.CompilerParams(needs_layout_passes=False),
        scratch_types=[
            pltpu.VMEM((2, F), jnp.float32),
            pltpu.VMEM((2, F), jnp.float32),
            pltpu.VMEM((F + 2 * L,), jnp.int32),
            pltpu.VMEM((F + 2 * L,), jnp.float32),
            pltpu.VMEM((96,), jnp.int32),
            pltpu.VMEM((96,), jnp.float32),
            pltpu.VMEM((96,), jnp.int32),
            pltpu.VMEM((2, 160), jnp.int32),
            pltpu.VMEM((16 * 256,), jnp.int32),
            pltpu.VMEM((L,), jnp.int32),
            pltpu.SemaphoreType.DMA((2,)),
            pltpu.SemaphoreType.DMA((2,)),
        ],
    )
    return fn(post)


def _run(x, W_enc, b_enc, W_dec, b_dec, *, k, tb, tf, tb3, tfk,
         interpret=False):
    B, D = x.shape
    F = W_enc.shape[0]

    post = pl.pallas_call(
        _encode_body,
        grid=(F // tf, B // tb),
        in_specs=[
            pl.BlockSpec((tb, D), lambda f, b: (b, 0)),
            pl.BlockSpec((tf, D), lambda f, b: (f, 0)),
            pl.BlockSpec((tf,), lambda f, b: (f,)),
            pl.BlockSpec((D,), lambda f, b: (0,)),
        ],
        out_specs=pl.BlockSpec((tb, tf), lambda f, b: (b, f)),
        out_shape=jax.ShapeDtypeStruct((B, F), jnp.float32),
        interpret=interpret,
    )(x, W_enc, b_enc, b_dec)

    masked = _sc_topk_mask(post, k)

    x_hat = pl.pallas_call(
        _decode_body,
        grid=(B // tb3, F // tfk),
        in_specs=[
            pl.BlockSpec((tb3, tfk), lambda i, kk: (i, kk)),
            pl.BlockSpec((D, tfk), lambda i, kk: (0, kk)),
            pl.BlockSpec((D,), lambda i, kk: (0,)),
        ],
        out_specs=pl.BlockSpec((tb3, D), lambda i, kk: (i, 0)),
        out_shape=jax.ShapeDtypeStruct((B, D), jnp.float32),
        compiler_params=pltpu.CompilerParams(
            dimension_semantics=("arbitrary", "arbitrary")),
        interpret=interpret,
    )(masked, W_dec, b_dec)
    return x_hat


def kernel(x, W_enc, b_enc, W_dec, b_dec):
    return _run(x, W_enc, b_enc, W_dec, b_dec,
                k=TOPK, tb=512, tf=2048, tb3=512, tfk=2048)


# speedup vs baseline: 3.8616x; 1.4188x over previous
import functools

import jax
import jax.numpy as jnp
from jax import lax
from jax.experimental import pallas as pl
from jax.experimental.pallas import tpu as pltpu
from jax.experimental.pallas import tpu_sc as plsc

TOPK = 64
L = 16


def _encode_body(x_ref, w_ref, benc_ref, bdec_ref, out_ref):
    x = x_ref[...] - bdec_ref[...][None, :]
    pre = jax.lax.dot_general(
        x, w_ref[...], (((1,), (1,)), ((), ())),
        preferred_element_type=jnp.float32)
    out_ref[...] = jnp.maximum(pre + benc_ref[...][None, :], 0.0)


def _decode_body(m_ref, w_ref, bdec_ref, out_ref):
    kk = pl.program_id(1)

    @pl.when(kk == 0)
    def _init():
        out_ref[...] = jnp.broadcast_to(bdec_ref[...][None, :], out_ref.shape)

    out_ref[...] += jax.lax.dot_general(
        m_ref[...], w_ref[...], (((1,), (1,)), ((), ())),
        preferred_element_type=jnp.float32)


def _pcnt(mask):
    return jnp.sum(mask.astype(jnp.int32))


def _sc_mask_body(rows_w, nvec, k, post_hbm, out_hbm, row_in, out_buf,
                  cand, cand_val, gt_idx, gt_val, eq_idx, usel, hist, tmp,
                  sem_in, sem_out):
    wid = lax.axis_index("s") * 2 + lax.axis_index("c")
    base = wid * rows_w
    lanes = lax.iota(jnp.int32, L)
    ones = jnp.ones((L,), jnp.int32)
    zf = jnp.zeros((L,), jnp.float32)
    zi = jnp.zeros((L,), jnp.int32)
    UN = 8

    def _zero(i, _):
        out_buf[0, pl.ds(i * L, L)] = zf
        out_buf[1, pl.ds(i * L, L)] = zf
        return 0
    lax.fori_loop(0, nvec, _zero, 0)

    def _zeroh(i, _):
        hist[pl.ds(i * L, L)] = zi
        return 0
    lax.fori_loop(0, 256, _zeroh, 0)

    pltpu.make_async_copy(post_hbm.at[base], row_in.at[0], sem_in.at[0]).start()
    pltpu.make_async_copy(post_hbm.at[base + 1], row_in.at[1],
                          sem_in.at[1]).start()

    def _scalar(v):
        return v[0]

    def _process(r, ib, tg):
        row = base + r
        pltpu.make_async_copy(post_hbm.at[row], row_in.at[ib],
                              sem_in.at[ib]).wait()
        ibv = jnp.full((L,), ib, jnp.int32)

        def _extract(t0s):
            @plsc.parallel_loop(0, nvec, unroll=UN, carry=zi)
            def _ex(i, cnt):
                vals = row_in[ib, pl.ds(i * L, L)]
                bits = lax.bitcast_convert_type(vals, jnp.int32)
                m = bits >= t0s
                dest = cnt + plsc.cumsum(m.astype(jnp.int32)) - 1
                plsc.store_scatter(cand, [dest], i * L + lanes, mask=m)
                plsc.store_scatter(cand_val, [dest], vals, mask=m)
                return cnt + plsc.all_reduce_population_count(m)
            return _ex

        c_n_v = _extract(tg)
        c_n0 = _scalar(c_n_v)

        def _fallback(_):
            @plsc.parallel_loop(0, nvec, unroll=UN)
            def _hist(i):
                bits = lax.bitcast_convert_type(
                    row_in[ib, pl.ds(i * L, L)], jnp.int32)
                plsc.addupdate_scatter(
                    hist, [lanes * 256 + (bits >> 23)], ones)

            def _scan(cc, carry):
                tot_above, b = carry
                c = 15 - cc
                tot = zi
                for lane in range(L):
                    off = lane * 256 + c * L
                    tot = tot + hist[pl.ds(off, L)]
                    hist[pl.ds(off, L)] = zi
                suf = lax.rev(plsc.cumsum(lax.rev(tot, (0,))), (0,)) + tot_above
                p = jnp.sum((suf >= k).astype(jnp.int32)) - 1
                b = jnp.where((b < 0) & (p >= 0), c * L + p, b)
                return tot_above + jnp.sum(tot), b
            _, b = lax.fori_loop(0, 16, _scan, (jnp.int32(0), jnp.int32(-1)))
            return _extract(jnp.full((L,), b << 23, jnp.int32))

        c_n_v = lax.cond((c_n0 < k) | (c_n0 > 1024), _fallback,
                         lambda _: c_n_v, 0)
        c_n = _scalar(c_n_v)
        nv = (c_n + L - 1) // L
        cand[pl.ds(c_n, L)] = zi
        cand_val[pl.ds(c_n, L)] = zf

        def _count_ge(t):
            @plsc.parallel_loop(0, nv, unroll=4, carry=zi)
            def _cnt(j, acc):
                bits = lax.bitcast_convert_type(
                    cand_val[pl.ds(j * L, L)], jnp.int32)
                return acc + plsc.all_reduce_population_count(bits >= t)
            return _cnt

        tlo = zi
        for bit in range(30, -1, -1):
            t = tlo | (1 << bit)
            cnt = _count_ge(t)
            tlo = jnp.where(cnt >= k, t, tlo)
        t_k = tlo

        def _sel(j, carry):
            g, e = carry
            idxv = cand[pl.ds(j * L, L)]
            vals = cand_val[pl.ds(j * L, L)]
            bits = lax.bitcast_convert_type(vals, jnp.int32)
            mgt = bits > t_k
            meq = bits == t_k
            destg = g + plsc.cumsum(mgt.astype(jnp.int32)) - 1
            plsc.store_scatter(gt_idx, [destg], idxv, mask=mgt)
            plsc.store_scatter(gt_val, [destg], vals, mask=mgt)
            deste = jnp.minimum(
                e + plsc.cumsum(meq.astype(jnp.int32)) - 1, 72 + lanes)
            plsc.store_scatter(eq_idx, [deste], idxv, mask=meq)
            return (g + plsc.all_reduce_population_count(mgt),
                    e + plsc.all_reduce_population_count(meq))
        g_v, _ = lax.fori_loop(0, nv, _sel, (zi, zi))
        g_n = _scalar(g_v)

        @pl.when(r >= 2)
        def _reclaim():
            pltpu.make_async_copy(out_buf.at[ib], out_hbm.at[row - 2],
                                  sem_out.at[ib]).wait()
            for j in range(TOPK // L):
                m = j * L + lanes < TOPK
                uv = jnp.where(m, usel[ib, pl.ds(j * L, L)], 0)
                plsc.store_scatter(out_buf, [ibv, uv], zf, mask=m)

        tf = lax.bitcast_convert_type(t_k, jnp.float32)
        for j in range(TOPK // L):
            pos = j * L + lanes
            mg = pos < g_v
            gi = jnp.where(mg, gt_idx[pl.ds(j * L, L)], 0)
            gv = gt_val[pl.ds(j * L, L)]
            plsc.store_scatter(out_buf, [ibv, gi], gv, mask=mg)
            me = pos < (k - g_v)
            ei = jnp.where(me, eq_idx[pl.ds(j * L, L)], 0)
            plsc.store_scatter(out_buf, [ibv, ei], tf, mask=me)
            usel[ib, pl.ds(j * L, L)] = gt_idx[pl.ds(j * L, L)]
        for j in range(TOPK // L):
            usel[ib, pl.ds(g_n + j * L, L)] = eq_idx[pl.ds(j * L, L)]

        pltpu.make_async_copy(out_buf.at[ib], out_hbm.at[row],
                              sem_out.at[ib]).start()

        @pl.when(r + 2 < rows_w)
        def _refill():
            pltpu.make_async_copy(post_hbm.at[row + 2], row_in.at[ib],
                                  sem_in.at[ib]).start()

        return jnp.maximum(t_k - (1 << 21), 0)

    def _pair(q, tg):
        tg = _process(2 * q, 0, tg)
        tg = _process(2 * q + 1, 1, tg)
        return tg
    lax.fori_loop(0, rows_w // 2, _pair, zi)

    pltpu.make_async_copy(out_buf.at[0], out_hbm.at[base + rows_w - 2],
                          sem_out.at[0]).wait()
    pltpu.make_async_copy(out_buf.at[1], out_hbm.at[base + rows_w - 1],
                          sem_out.at[1]).wait()


def _sc_topk_mask(post, k):
    B, F = post.shape
    info = plsc.get_sparse_core_info()
    nw = info.num_cores * info.num_subcores
    rows_w = B // nw
    nvec = F // L
    mesh = plsc.VectorSubcoreMesh(core_axis_name="c", subcore_axis_name="s")
    fn = pl.kernel(
        functools.partial(_sc_mask_body, rows_w, nvec, k),
        out_type=jax.ShapeDtypeStruct((B, F), jnp.float32),
        mesh=mesh,
        compiler_params=pltpu.CompilerParams(needs_layout_passes=False),
        scratch_types=[
            pltpu.VMEM((2, F), jnp.float32),
            pltpu.VMEM((2, F), jnp.float32),
            pltpu.VMEM((F + 2 * L,), jnp.int32),
            pltpu.VMEM((F + 2 * L,), jnp.float32),
            pltpu.VMEM((96,), jnp.int32),
            pltpu.VMEM((96,), jnp.float32),
            pltpu.VMEM((96,), jnp.int32),
            pltpu.VMEM((2, 160), jnp.int32),
            pltpu.VMEM((16 * 256,), jnp.int32),
            pltpu.VMEM((L,), jnp.int32),
            pltpu.SemaphoreType.DMA((2,)),
            pltpu.SemaphoreType.DMA((2,)),
        ],
    )
    return fn(post)


def _run(x, W_enc, b_enc, W_dec, b_dec, *, k, tb, tf, tb3, tfk,
         interpret=False, nchunk=4):
    B, D = x.shape
    F = W_enc.shape[0]
    cb = B // nchunk
    outs = []
    for c in range(nchunk):
        xc = lax.slice(x, (c * cb, 0), ((c + 1) * cb, D))
        outs.append(_run_chunk(xc, W_enc, b_enc, W_dec, b_dec, k=k, tb=tb,
                               tf=tf, tb3=tb3, tfk=tfk, interpret=interpret))
    return jnp.concatenate(outs, axis=0)


def _run_chunk(x, W_enc, b_enc, W_dec, b_dec, *, k, tb, tf, tb3, tfk,
               interpret=False):
    B, D = x.shape
    F = W_enc.shape[0]

    post = pl.pallas_call(
        _encode_body,
        grid=(F // tf, B // tb),
        in_specs=[
            pl.BlockSpec((tb, D), lambda f, b: (b, 0)),
            pl.BlockSpec((tf, D), lambda f, b: (f, 0)),
            pl.BlockSpec((tf,), lambda f, b: (f,)),
            pl.BlockSpec((D,), lambda f, b: (0,)),
        ],
        out_specs=pl.BlockSpec((tb, tf), lambda f, b: (b, f)),
        out_shape=jax.ShapeDtypeStruct((B, F), jnp.float32),
        interpret=interpret,
    )(x, W_enc, b_enc, b_dec)

    masked = _sc_topk_mask(post, k)

    x_hat = pl.pallas_call(
        _decode_body,
        grid=(B // tb3, F // tfk),
        in_specs=[
            pl.BlockSpec((tb3, tfk), lambda i, kk: (i, kk)),
            pl.BlockSpec((D, tfk), lambda i, kk: (0, kk)),
            pl.BlockSpec((D,), lambda i, kk: (0,)),
        ],
        out_specs=pl.BlockSpec((tb3, D), lambda i, kk: (i, 0)),
        out_shape=jax.ShapeDtypeStruct((B, D), jnp.float32),
        compiler_params=pltpu.CompilerParams(
            dimension_semantics=("arbitrary", "arbitrary")),
        interpret=interpret,
    )(masked, W_dec, b_dec)
    return x_hat


def kernel(x, W_enc, b_enc, W_dec, b_dec):
    return _run(x, W_enc, b_enc, W_dec, b_dec,
                k=TOPK, tb=512, tf=2048, tb3=512, tfk=2048)
